# baseline (speedup 1.0000x reference)
import functools

import jax
import jax.numpy as jnp
from jax.experimental import pallas as pl
from jax.experimental.pallas import tpu as pltpu

HID = 128
HEADS = 4
NUM_HERBS = 4096
NUM_SYMPTOMS = 2048
NUM_INGREDIENTS = 8192
NUM_HYPEREDGES = 2048
BATCH = 256


def _head_body(x_ref, fs_ref, fh_ref, w1_ref, b1_ref, w2_ref, b2_ref,
               out_ref, logits_ref):
    j = pl.program_id(0)

    @pl.when(j == 0)
    def _():
        semb = jnp.dot(x_ref[...], fs_ref[...],
                       preferred_element_type=jnp.float32)
        logits_ref[...] = jax.lax.dot_general(
            semb, fh_ref[...], (((1,), (1,)), ((), ())),
            preferred_element_type=jnp.float32)
        out_ref[...] = jnp.broadcast_to(b2_ref[...], out_ref.shape)

    hid = jnp.dot(logits_ref[...], w1_ref[...],
                  preferred_element_type=jnp.float32) + b1_ref[...]
    hid = jnp.maximum(hid, 0.0)
    out_ref[...] += jnp.dot(hid, w2_ref[...],
                            preferred_element_type=jnp.float32)


def _head(x, final_sym, final_herb, w1, b1, w2, b2):
    BK = 512
    nk = w1.shape[1] // BK
    b1r = b1.reshape(1, -1)
    b2r = b2.reshape(1, -1)
    return pl.pallas_call(
        _head_body,
        grid=(nk,),
        in_specs=[
            pl.BlockSpec((BATCH, NUM_SYMPTOMS), lambda j: (0, 0)),
            pl.BlockSpec((NUM_SYMPTOMS, HID), lambda j: (0, 0)),
            pl.BlockSpec((NUM_HERBS, HID), lambda j: (0, 0)),
            pl.BlockSpec((NUM_HERBS, BK), lambda j: (0, j)),
            pl.BlockSpec((1, BK), lambda j: (0, j)),
            pl.BlockSpec((BK, NUM_HERBS), lambda j: (j, 0)),
            pl.BlockSpec((1, NUM_HERBS), lambda j: (0, 0)),
        ],
        out_specs=pl.BlockSpec((BATCH, NUM_HERBS), lambda j: (0, 0)),
        out_shape=jax.ShapeDtypeStruct((BATCH, NUM_HERBS), jnp.float32),
        scratch_shapes=[pltpu.VMEM((BATCH, NUM_HERBS), jnp.float32)],
    )(x, final_sym, final_herb, w1, b1r, w2, b2r)


def _gat(x, edge_index, heads, out_ch, concat, W, a_src, a_dst, b):
    N = x.shape[0]
    loops = jnp.arange(N, dtype=edge_index.dtype)
    src = jnp.concatenate([edge_index[0], loops])
    dst = jnp.concatenate([edge_index[1], loops])
    h = (x @ W).reshape(N, heads, out_ch)
    asrc = (h * a_src).sum(-1)
    adst = (h * a_dst).sum(-1)
    e = jax.nn.leaky_relu(asrc[src] + adst[dst], 0.2)
    m = jax.ops.segment_max(e, dst, num_segments=N)
    ex = jnp.exp(e - m[dst])
    den = jax.ops.segment_sum(ex, dst, num_segments=N)
    alpha = ex / (den[dst] + 1e-16)
    out = jax.ops.segment_sum(h[src] * alpha[:, :, None], dst, num_segments=N)
    out = out.reshape(N, heads * out_ch) if concat else out.mean(axis=1)
    return out + b


def _hyperconv(x, edge_index, W, b):
    node = edge_index[0]
    he = edge_index[1]
    N = x.shape[0]
    xw = x @ W
    ones_n = jnp.ones(node.shape[0], dtype=jnp.float32)
    D = jax.ops.segment_sum(ones_n, node, num_segments=N)
    Bdeg = jax.ops.segment_sum(ones_n, he, num_segments=NUM_HYPEREDGES)
    Dinv = jnp.where(D > 0, 1.0 / jnp.maximum(D, 1.0), 0.0)
    Binv = jnp.where(Bdeg > 0, 1.0 / jnp.maximum(Bdeg, 1.0), 0.0)
    msg = jax.ops.segment_sum(xw[node], he, num_segments=NUM_HYPEREDGES) * Binv[:, None]
    out = jax.ops.segment_sum(msg[he], node, num_segments=N) * Dinv[:, None]
    return out + b


def _scatter_mean(src, idx, dim_size):
    s = jax.ops.segment_sum(src, idx, num_segments=dim_size)
    c = jax.ops.segment_sum(jnp.ones(src.shape[0], dtype=src.dtype), idx,
                            num_segments=dim_size)
    return s / jnp.maximum(c, 1.0)[:, None]


def kernel(x, herb_x, symptom_x, cross_x, hyper_x, params, herb_edge_index,
           symptom_edge_index, cross_edge_index, hyper_edge_index,
           hyper_edge_mapping):
    p = params
    hx = jax.nn.elu(_gat(herb_x, herb_edge_index, HEADS, HID, True, **p['herb_gat1']))
    hx = jax.nn.elu(_gat(hx, herb_edge_index, 1, HID, True, **p['herb_gat2']))
    sx = jax.nn.elu(_gat(symptom_x, symptom_edge_index, HEADS, HID, True, **p['sym_gat1']))
    sx = jax.nn.elu(_gat(sx, symptom_edge_index, 1, HID, True, **p['sym_gat2']))
    cx = jax.nn.elu(_gat(cross_x, cross_edge_index, HEADS, HID, True, **p['cross_gat1']))
    cx = jax.nn.elu(_gat(cx, cross_edge_index, 1, HID, True, **p['cross_gat2']))
    hy = jax.nn.relu(_hyperconv(hyper_x, hyper_edge_index, p['hyper1']['W'], p['hyper1']['b']))
    hy = _hyperconv(hy, hyper_edge_index, p['hyper2']['W'], p['hyper2']['b'])
    herb_from_hyper = _scatter_mean(hy, hyper_edge_mapping, NUM_HERBS)
    final_sym = sx + cx[:NUM_SYMPTOMS]
    final_herb = hx + cx[NUM_SYMPTOMS:] + herb_from_hyper
    return _head(x, final_sym, final_herb, p['pred_W1'], p['pred_b1'],
                 p['pred_W2'], p['pred_b2'])


# speedup vs baseline: 11.6069x; 11.6069x over previous
import functools

import jax
import jax.numpy as jnp
from jax import lax
from jax.experimental import pallas as pl
from jax.experimental.pallas import tpu as pltpu
from jax.experimental.pallas import tpu_sc as plsc

HID = 128
HEADS = 4
NUM_HERBS = 4096
NUM_SYMPTOMS = 2048
NUM_INGREDIENTS = 8192
NUM_HYPEREDGES = 2048
BATCH = 256

_B = 64

_SC_PARAMS = pltpu.CompilerParams(use_tc_tiling_on_sc=False,
                                  needs_layout_passes=False)


def _mesh():
    return plsc.VectorSubcoreMesh(core_axis_name="c", subcore_axis_name="s",
                                  num_cores=2, num_subcores=16)


def _mm_body(x_ref, w_ref, o_ref):
    o_ref[...] = jnp.dot(x_ref[...], w_ref[...],
                         preferred_element_type=jnp.float32)


def _mm(x, w):
    n, _ = x.shape
    m = w.shape[1]
    return pl.pallas_call(
        _mm_body,
        out_shape=jax.ShapeDtypeStruct((n, m), jnp.float32),
    )(x, w)


@functools.lru_cache(maxsize=None)
def _make_gat(N, Etot, NH, CW):
    B = _B
    E16 = Etot // 16
    NB = E16 // B
    NR = N // 16
    NRB = NR // B
    NP = max(CW // 64, 1)
    PW = CW // NP
    NCB = PW // 16
    HL16 = (CW // NH) // 16

    @functools.partial(
        pl.kernel,
        out_type=jax.ShapeDtypeStruct((2 * NP * N, PW), jnp.float32),
        mesh=_mesh(),
        compiler_params=_SC_PARAMS,
        scratch_types=[
            pltpu.VMEM((N * NH,), jnp.float32),
            pltpu.VMEM((N * NH,), jnp.float32),
            pltpu.VMEM((NB, B), jnp.int32),
            pltpu.VMEM((NH * E16,), jnp.float32),
            pltpu.VMEM((B, 16), jnp.float32),
            pltpu.VMEM((N * NH,), jnp.float32),
            pltpu.VMEM((NR, 16), jnp.float32),
            pltpu.VMEM((B, PW), jnp.float32),
            pltpu.VMEM((B, PW), jnp.float32),
            pltpu.VMEM((B,), jnp.int32),
            pltpu.VMEM((B,), jnp.int32),
            pltpu.VMEM((PW,), jnp.float32),
            pltpu.VMEM((NH * B,), jnp.float32),
            pltpu.VMEM_SHARED((N, 16), jnp.float32),
            pltpu.VMEM_SHARED((N * NH,), jnp.float32),
            pltpu.VMEM_SHARED((N, PW), jnp.float32),
            pltpu.SemaphoreType.DMA,
            pltpu.SemaphoreType.DMA,
        ],
    )
    def gat(h_hbm, asc_hbm, adc_hbm, srcp_hbm, dstb_hbm, b_hbm, out_hbm,
            asrc_t, adst_t, dstb, exl, exrow, rden_t, denrow, rows_a,
            rows_b, srca, srcb, bias_t, alpha_t, den_sh, rden_sh, acc_sh,
            sem_a, sem_b):
        c = lax.axis_index("c")
        s = lax.axis_index("s")
        iota = lax.iota(jnp.int32, 16)
        zero16 = jnp.zeros((16,), jnp.float32)
        toff = s * E16
        rbase = s * NR

        pltpu.sync_copy(asc_hbm.at[c], asrc_t)
        pltpu.sync_copy(adc_hbm.at[c], adst_t)
        pltpu.sync_copy(dstb_hbm.at[s], dstb)

        for i in range(B):
            exrow[i, :] = zero16

        def _zrow(i, carry):
            ii = jnp.full((16,), i, jnp.int32)
            for j in range(NCB):
                plsc.store_scatter(rows_a, [ii, j * 16 + iota], zero16)
            return carry
        lax.fori_loop(0, B, _zrow, 0)

        for q in range(NRB):
            pltpu.sync_copy(exrow, den_sh.at[pl.ds(rbase + q * B, B)])
        plsc.subcore_barrier()

        def _p1(blk, carry):
            pltpu.sync_copy(srcp_hbm.at[pl.ds(toff + blk * B, B)], srca)
            ebase = blk * B
            for sub in range(4):
                off = sub * 16 + iota
                s16 = plsc.load_gather(srca, [off])
                d16 = plsc.load_gather(
                    dstb, [jnp.full((16,), 0, jnp.int32) + blk, off])
                for hh in range(NH):
                    av = plsc.load_gather(asrc_t, [s16 * NH + hh])
                    bv = plsc.load_gather(adst_t, [d16 * NH + hh])
                    e = av + bv
                    e = jnp.where(e > 0, e, e * jnp.float32(0.2))
                    ex = jnp.exp(e)
                    plsc.store_scatter(
                        exl, [ebase + (hh * E16 + sub * 16) + iota], ex)
                    plsc.store_scatter(
                        exrow, [off, jnp.full((16,), hh, jnp.int32)], ex)
            pltpu.sync_copy(exrow, den_sh.at[dstb.at[blk]], add=True)
            return carry
        lax.fori_loop(0, NB, _p1, 0)
        plsc.subcore_barrier()

        pltpu.sync_copy(den_sh.at[pl.ds(rbase, NR)], denrow)
        for i in range(NR // 16):
            for hh in range(NH):
                d = plsc.load_gather(
                    denrow, [i * 16 + iota, jnp.full((16,), hh, jnp.int32)])
                r = 1.0 / (d + jnp.float32(1e-16))
                plsc.store_scatter(rden_t, [(i * 16 + iota) * NH + hh], r)
        pltpu.sync_copy(rden_t.at[pl.ds(0, NR * NH)],
                        rden_sh.at[pl.ds(rbase * NH, NR * NH)])
        plsc.subcore_barrier()
        pltpu.sync_copy(rden_sh, rden_t)

        for ps in range(NP):
            base = c * NP + ps
            gN = base * N

            def _azrow(i, carry):
                ii = jnp.full((16,), i, jnp.int32)
                for j in range(NCB):
                    plsc.store_scatter(rows_a, [ii, j * 16 + iota], zero16)
                return carry
            lax.fori_loop(0, B, _azrow, 0)
            for q in range(NRB):
                pltpu.sync_copy(rows_a, acc_sh.at[pl.ds(rbase + q * B, B)])
            plsc.subcore_barrier()

            def _loadsrc(blk, sref):
                pltpu.sync_copy(srcp_hbm.at[pl.ds(toff + blk * B, B)], sref)
                for q in range(B // 16):
                    v = plsc.load_gather(sref, [q * 16 + iota])
                    plsc.store_scatter(sref, [q * 16 + iota], v + gN)

            def _scale_scatter(blk, rows):
                for sub in range(4):
                    off = sub * 16 + iota
                    d16 = plsc.load_gather(
                        dstb, [jnp.full((16,), 0, jnp.int32) + blk, off])
                    for hh in range(NH):
                        ex16 = plsc.load_gather(
                            exl, [blk * B + (hh * E16 + sub * 16) + iota])
                        rd16 = plsc.load_gather(rden_t, [d16 * NH + hh])
                        plsc.store_scatter(
                            alpha_t, [hh * B + sub * 16 + iota], ex16 * rd16)

                def _srow(i, carry):
                    ii = jnp.full((16,), i, jnp.int32)
                    heads = sorted({(ps * NCB + j) // HL16
                                    for j in range(NCB)})
                    bcs = {hh: plsc.load_gather(
                        alpha_t, [jnp.full((16,), hh * B, jnp.int32) + i])
                        for hh in heads}
                    for j in range(NCB):
                        hh = (ps * NCB + j) // HL16
                        v = plsc.load_gather(rows, [ii, j * 16 + iota])
                        plsc.store_scatter(rows, [ii, j * 16 + iota],
                                           v * bcs[hh])
                    return carry
                lax.fori_loop(0, B, _srow, 0)
                pltpu.sync_copy(rows, acc_sh.at[dstb.at[blk]], add=True)

            def _p2(p, carry):
                blk_a = 2 * p
                blk_b = 2 * p + 1
                _loadsrc(blk_a, srca)
                da = pltpu.async_copy(h_hbm.at[srca], rows_a, sem_a)
                _loadsrc(blk_b, srcb)
                db = pltpu.async_copy(h_hbm.at[srcb], rows_b, sem_b)
                da.wait()
                _scale_scatter(blk_a, rows_a)
                db.wait()
                _scale_scatter(blk_b, rows_b)
                return carry
            lax.fori_loop(0, NB // 2, _p2, 0)
            plsc.subcore_barrier()

            pltpu.sync_copy(b_hbm.at[base], bias_t)
            for q in range(NRB):
                row0 = rbase + q * B
                pltpu.sync_copy(acc_sh.at[pl.ds(row0, B)], rows_a)

                def _orow(i, carry):
                    ii = jnp.full((16,), i, jnp.int32)
                    for j in range(NCB):
                        v = plsc.load_gather(rows_a, [ii, j * 16 + iota])
                        v = v + bias_t[pl.ds(j * 16, 16)]
                        v = jnp.where(v > 0, v, jnp.exp(v) - 1.0)
                        plsc.store_scatter(rows_a, [ii, j * 16 + iota], v)
                    return carry
                lax.fori_loop(0, B, _orow, 0)
                pltpu.sync_copy(rows_a, out_hbm.at[pl.ds(gN + row0, B)])
            plsc.subcore_barrier()

    return gat


def _gat_layer(x, src, dst, H, p):
    N = x.shape[0]
    Etot = src.shape[0]
    W, a_src, a_dst, b = p['W'], p['a_src'], p['a_dst'], p['b']
    din = W.shape[0]
    Asrc = jnp.einsum('dhk,hk->dh', W.reshape(din, H, HID), a_src[0])
    Adst = jnp.einsum('dhk,hk->dh', W.reshape(din, H, HID), a_dst[0])
    pad = jnp.zeros((din, 128 - 2 * H), jnp.float32)
    Wext = jnp.concatenate([W, Asrc, Adst, pad], axis=1)
    ho = _mm(x, Wext)
    h = ho[:, :H * HID]
    asrc = ho[:, H * HID:H * HID + H]
    adst = ho[:, H * HID + H:H * HID + 2 * H]

    CW = (H * HID) // 2
    NH = max(H // 2, 1)
    NP = max(CW // 64, 1)
    PW = CW // NP
    nparts = 2 * NP
    h_sc = jnp.concatenate(
        [h[:, i * PW:(i + 1) * PW] for i in range(nparts)], axis=0)
    if H > 1:
        asc = jnp.stack([asrc[:, :NH].reshape(-1), asrc[:, NH:].reshape(-1)])
        adc = jnp.stack([adst[:, :NH].reshape(-1), adst[:, NH:].reshape(-1)])
    else:
        asc = jnp.stack([asrc[:, 0], asrc[:, 0]])
        adc = jnp.stack([adst[:, 0], adst[:, 0]])
    b_sc = jnp.stack([b[i * PW:(i + 1) * PW] for i in range(nparts)])
    srcp = jnp.concatenate([src, jnp.zeros((_B,), jnp.int32)])
    dstb = dst.reshape(16, -1, _B)

    out2 = _make_gat(N, Etot, NH, CW)(h_sc, asc, adc, srcp, dstb, b_sc)
    return jnp.concatenate(
        [out2[i * N:(i + 1) * N] for i in range(nparts)], axis=1)


@functools.lru_cache(maxsize=None)
def _make_hyper1():
    B = _B
    NI = NUM_INGREDIENTS
    NE = NUM_HYPEREDGES
    E = 65536
    E16 = E // 16
    NB = E16 // B
    NRI = NI // 16
    NRE = NE // 16
    CW = 64
    NCB = CW // 16

    @functools.partial(
        pl.kernel,
        out_type=[jax.ShapeDtypeStruct((2 * NE, CW), jnp.float32),
                  jax.ShapeDtypeStruct((2 * NI,), jnp.float32)],
        mesh=_mesh(),
        compiler_params=_SC_PARAMS,
        scratch_types=[
            pltpu.VMEM((NB, B), jnp.int32),
            pltpu.VMEM((NB, B), jnp.int32),
            pltpu.VMEM((B, CW), jnp.float32),
            pltpu.VMEM((B, CW), jnp.float32),
            pltpu.VMEM((B,), jnp.int32),
            pltpu.VMEM((B,), jnp.int32),
            pltpu.VMEM((B, 16), jnp.float32),
            pltpu.VMEM((NRE, 16), jnp.float32),
            pltpu.VMEM((NRI, 16), jnp.float32),
            pltpu.VMEM((NRI,), jnp.float32),
            pltpu.VMEM_SHARED((NE, CW), jnp.float32),
            pltpu.VMEM_SHARED((NI, 16), jnp.float32),
            pltpu.VMEM_SHARED((NE, 16), jnp.float32),
            pltpu.SemaphoreType.DMA,
            pltpu.SemaphoreType.DMA,
        ],
    )
    def hyper1(xw_hbm, nodep_hbm, nodeb_hbm, heb_hbm, msg_hbm, dinv_hbm,
               nodeb, heb, rows_a, rows_b, ia, ib, onesrow, degErow,
               degNrow, dinv_v, msg_sh, degN_sh, degE_sh, sem_a, sem_b):
        c = lax.axis_index("c")
        s = lax.axis_index("s")
        iota = lax.iota(jnp.int32, 16)
        zero16 = jnp.zeros((16,), jnp.float32)
        one0 = jnp.where(iota == 0, jnp.float32(1.0), jnp.float32(0.0))
        toff = s * E16

        pltpu.sync_copy(nodeb_hbm.at[s], nodeb)
        pltpu.sync_copy(heb_hbm.at[s], heb)

        for i in range(B):
            onesrow[i, :] = zero16

        def _zrow(i, carry):
            ii = jnp.full((16,), i, jnp.int32)
            for j in range(NCB):
                plsc.store_scatter(rows_a, [ii, j * 16 + iota], zero16)
            return carry
        lax.fori_loop(0, B, _zrow, 0)

        for q in range(NRI // B):
            pltpu.sync_copy(onesrow, degN_sh.at[pl.ds(s * NRI + q * B, B)])
        for q in range(NRE // B):
            pltpu.sync_copy(onesrow, degE_sh.at[pl.ds(s * NRE + q * B, B)])
            pltpu.sync_copy(rows_a, msg_sh.at[pl.ds(s * NRE + q * B, B)])
        for i in range(B):
            onesrow[i, :] = one0
        plsc.subcore_barrier()

        def _loadidx(hbm, blk, sref, shift):
            pltpu.sync_copy(hbm.at[pl.ds(toff + blk * B, B)], sref)
            for q in range(B // 16):
                v = plsc.load_gather(sref, [q * 16 + iota])
                plsc.store_scatter(sref, [q * 16 + iota], v + shift)

        def _pa(p, carry):
            blk_a = 2 * p
            blk_b = 2 * p + 1
            _loadidx(nodep_hbm, blk_a, ia, c * NI)
            da = pltpu.async_copy(xw_hbm.at[ia], rows_a, sem_a)
            _loadidx(nodep_hbm, blk_b, ib, c * NI)
            db = pltpu.async_copy(xw_hbm.at[ib], rows_b, sem_b)
            da.wait()
            pltpu.sync_copy(rows_a, msg_sh.at[heb.at[blk_a]], add=True)
            pltpu.sync_copy(onesrow, degN_sh.at[nodeb.at[blk_a]], add=True)
            pltpu.sync_copy(onesrow, degE_sh.at[heb.at[blk_a]], add=True)
            db.wait()
            pltpu.sync_copy(rows_b, msg_sh.at[heb.at[blk_b]], add=True)
            pltpu.sync_copy(onesrow, degN_sh.at[nodeb.at[blk_b]], add=True)
            pltpu.sync_copy(onesrow, degE_sh.at[heb.at[blk_b]], add=True)
            return carry
        lax.fori_loop(0, NB // 2, _pa, 0)
        plsc.subcore_barrier()

        pltpu.sync_copy(degE_sh.at[pl.ds(s * NRE, NRE)], degErow)
        for q in range(NRE // B):
            pltpu.sync_copy(msg_sh.at[pl.ds(s * NRE + q * B, B)], rows_a)

            def _mrow(i, carry):
                ii = jnp.full((16,), i, jnp.int32)
                d = plsc.load_gather(
                    degErow,
                    [jnp.full((16,), q * B, jnp.int32) + i,
                     jnp.zeros((16,), jnp.int32)])
                binv = jnp.where(d > 0, 1.0 / jnp.maximum(d, 1.0),
                                 jnp.float32(0.0))
                for j in range(NCB):
                    v = plsc.load_gather(rows_a, [ii, j * 16 + iota])
                    plsc.store_scatter(rows_a, [ii, j * 16 + iota], v * binv)
                return carry
            lax.fori_loop(0, B, _mrow, 0)
            pltpu.sync_copy(
                rows_a, msg_hbm.at[pl.ds(c * NE + s * NRE + q * B, B)])

        pltpu.sync_copy(degN_sh.at[pl.ds(s * NRI, NRI)], degNrow)
        for i in range(NRI // 16):
            d = plsc.load_gather(degNrow,
                                 [i * 16 + iota, jnp.zeros((16,), jnp.int32)])
            dinv = jnp.where(d > 0, 1.0 / jnp.maximum(d, 1.0),
                             jnp.float32(0.0))
            plsc.store_scatter(dinv_v, [i * 16 + iota], dinv)
        pltpu.sync_copy(dinv_v, dinv_hbm.at[pl.ds(c * NI + s * NRI, NRI)])

    return hyper1


@functools.lru_cache(maxsize=None)
def _make_hyper2(relu):
    B = _B
    NI = NUM_INGREDIENTS
    NE = NUM_HYPEREDGES
    E = 65536
    E16 = E // 16
    NB = E16 // B
    NRI = NI // 16
    CW = 64
    NCB = CW // 16

    @functools.partial(
        pl.kernel,
        out_type=jax.ShapeDtypeStruct((2 * NI, CW), jnp.float32),
        mesh=_mesh(),
        compiler_params=_SC_PARAMS,
        scratch_types=[
            pltpu.VMEM((NB, B), jnp.int32),
            pltpu.VMEM((B, CW), jnp.float32),
            pltpu.VMEM((B, CW), jnp.float32),
            pltpu.VMEM((B,), jnp.int32),
            pltpu.VMEM((B,), jnp.int32),
            pltpu.VMEM((NRI,), jnp.float32),
            pltpu.VMEM((CW,), jnp.float32),
            pltpu.VMEM_SHARED((NI, CW), jnp.float32),
            pltpu.SemaphoreType.DMA,
            pltpu.SemaphoreType.DMA,
        ],
    )
    def hyper2(msg_hbm, hep_hbm, nodeb_hbm, dinv_hbm, b_hbm, out_hbm,
               nodeb, rows_a, rows_b, ia, ib, dinv_v, bias_t, acc_sh,
               sem_a, sem_b):
        c = lax.axis_index("c")
        s = lax.axis_index("s")
        iota = lax.iota(jnp.int32, 16)
        zero16 = jnp.zeros((16,), jnp.float32)
        toff = s * E16

        pltpu.sync_copy(nodeb_hbm.at[s], nodeb)
        pltpu.sync_copy(b_hbm.at[c], bias_t)
        pltpu.sync_copy(dinv_hbm.at[pl.ds(c * NI + s * NRI, NRI)], dinv_v)

        def _zrow(i, carry):
            ii = jnp.full((16,), i, jnp.int32)
            for j in range(NCB):
                plsc.store_scatter(rows_a, [ii, j * 16 + iota], zero16)
            return carry
        lax.fori_loop(0, B, _zrow, 0)
        for q in range(NRI // B):
            pltpu.sync_copy(rows_a, acc_sh.at[pl.ds(s * NRI + q * B, B)])
        plsc.subcore_barrier()

        def _loadidx(hbm, blk, sref, shift):
            pltpu.sync_copy(hbm.at[pl.ds(toff + blk * B, B)], sref)
            for q in range(B // 16):
                v = plsc.load_gather(sref, [q * 16 + iota])
                plsc.store_scatter(sref, [q * 16 + iota], v + shift)

        def _pb(p, carry):
            blk_a = 2 * p
            blk_b = 2 * p + 1
            _loadidx(hep_hbm, blk_a, ia, c * NE)
            da = pltpu.async_copy(msg_hbm.at[ia], rows_a, sem_a)
            _loadidx(hep_hbm, blk_b, ib, c * NE)
            db = pltpu.async_copy(msg_hbm.at[ib], rows_b, sem_b)
            da.wait()
            pltpu.sync_copy(rows_a, acc_sh.at[nodeb.at[blk_a]], add=True)
            db.wait()
            pltpu.sync_copy(rows_b, acc_sh.at[nodeb.at[blk_b]], add=True)
            return carry
        lax.fori_loop(0, NB // 2, _pb, 0)
        plsc.subcore_barrier()

        for q in range(NRI // B):
            row0 = s * NRI + q * B
            pltpu.sync_copy(acc_sh.at[pl.ds(row0, B)], rows_a)

            def _orow(i, carry):
                ii = jnp.full((16,), i, jnp.int32)
                dinv = plsc.load_gather(
                    dinv_v, [jnp.full((16,), q * B, jnp.int32) + i])
                for j in range(NCB):
                    v = plsc.load_gather(rows_a, [ii, j * 16 + iota])
                    v = v * dinv + bias_t[pl.ds(j * 16, 16)]
                    if relu:
                        v = jnp.maximum(v, jnp.float32(0.0))
                    plsc.store_scatter(rows_a, [ii, j * 16 + iota], v)
                return carry
            lax.fori_loop(0, B, _orow, 0)
            pltpu.sync_copy(rows_a, out_hbm.at[pl.ds(c * NI + row0, B)])

    return hyper2


def _hyper_layer(xw_split, nodep, hep, nodeb, heb, b, relu):
    b_sc = jnp.stack([b[:64], b[64:]])
    msg, dinv = _make_hyper1()(xw_split, nodep, nodeb, heb)
    return _make_hyper2(relu)(msg, hep, nodeb, dinv, b_sc)


@functools.lru_cache(maxsize=None)
def _make_smean():
    B = _B
    NS = NUM_INGREDIENTS
    ND = NUM_HERBS
    S16 = NS // 16
    NB = S16 // B
    NRD = ND // 16
    CW = 64
    NCB = CW // 16

    @functools.partial(
        pl.kernel,
        out_type=jax.ShapeDtypeStruct((2 * ND, CW), jnp.float32),
        mesh=_mesh(),
        compiler_params=_SC_PARAMS,
        scratch_types=[
            pltpu.VMEM((NB, B), jnp.int32),
            pltpu.VMEM((B, CW), jnp.float32),
            pltpu.VMEM((B, 16), jnp.float32),
            pltpu.VMEM((NRD, 16), jnp.float32),
            pltpu.VMEM_SHARED((ND, 16), jnp.float32),
            pltpu.VMEM_SHARED((ND, CW), jnp.float32),
        ],
    )
    def smean(src_hbm, mapb_hbm, out_hbm, mapb, rows_a, onesrow, degrow,
              deg_sh, acc_sh):
        c = lax.axis_index("c")
        s = lax.axis_index("s")
        iota = lax.iota(jnp.int32, 16)
        zero16 = jnp.zeros((16,), jnp.float32)
        one0 = jnp.where(iota == 0, jnp.float32(1.0), jnp.float32(0.0))

        pltpu.sync_copy(mapb_hbm.at[s], mapb)
        for i in range(B):
            onesrow[i, :] = zero16

        def _zrow(i, carry):
            ii = jnp.full((16,), i, jnp.int32)
            for j in range(NCB):
                plsc.store_scatter(rows_a, [ii, j * 16 + iota], zero16)
            return carry
        lax.fori_loop(0, B, _zrow, 0)

        for q in range(NRD // B):
            pltpu.sync_copy(onesrow, deg_sh.at[pl.ds(s * NRD + q * B, B)])
            pltpu.sync_copy(rows_a, acc_sh.at[pl.ds(s * NRD + q * B, B)])
        for i in range(B):
            onesrow[i, :] = one0
        plsc.subcore_barrier()

        def _pa(blk, carry):
            pltpu.sync_copy(
                src_hbm.at[pl.ds(c * NS + s * S16 + blk * B, B)], rows_a)
            pltpu.sync_copy(rows_a, acc_sh.at[mapb.at[blk]], add=True)
            pltpu.sync_copy(onesrow, deg_sh.at[mapb.at[blk]], add=True)
            return carry
        lax.fori_loop(0, NB, _pa, 0)
        plsc.subcore_barrier()

        pltpu.sync_copy(deg_sh.at[pl.ds(s * NRD, NRD)], degrow)
        for q in range(NRD // B):
            row0 = s * NRD + q * B
            pltpu.sync_copy(acc_sh.at[pl.ds(row0, B)], rows_a)

            def _orow(i, carry):
                ii = jnp.full((16,), i, jnp.int32)
                d = plsc.load_gather(
                    degrow,
                    [jnp.full((16,), q * B, jnp.int32) + i,
                     jnp.zeros((16,), jnp.int32)])
                rinv = 1.0 / jnp.maximum(d, jnp.float32(1.0))
                for j in range(NCB):
                    v = plsc.load_gather(rows_a, [ii, j * 16 + iota])
                    plsc.store_scatter(rows_a, [ii, j * 16 + iota], v * rinv)
                return carry
            lax.fori_loop(0, B, _orow, 0)
            pltpu.sync_copy(rows_a, out_hbm.at[pl.ds(c * ND + row0, B)])

    return smean


def _head_body(x_ref, sx_ref, cxs_ref, hx_ref, cxh_ref, hfh_ref, w1_ref,
               b1_ref, w2_ref, b2_ref, out_ref, logits_ref):
    j = pl.program_id(0)

    @pl.when(j == 0)
    def _():
        fs = sx_ref[...] + cxs_ref[...]
        fh = hx_ref[...] + cxh_ref[...] + hfh_ref[...]
        semb = jnp.dot(x_ref[...], fs, preferred_element_type=jnp.float32)
        logits_ref[...] = lax.dot_general(
            semb, fh, (((1,), (1,)), ((), ())),
            preferred_element_type=jnp.float32)
        out_ref[...] = jnp.broadcast_to(b2_ref[...], out_ref.shape)

    hid = jnp.dot(logits_ref[...], w1_ref[...],
                  preferred_element_type=jnp.float32) + b1_ref[...]
    hid = jnp.maximum(hid, 0.0)
    out_ref[...] += jnp.dot(hid, w2_ref[...],
                            preferred_element_type=jnp.float32)


def _head(x, sx, cxs, hx, cxh, hfh, w1, b1, w2, b2):
    BK = 512
    nk = w1.shape[1] // BK
    b1r = b1.reshape(1, -1)
    b2r = b2.reshape(1, -1)
    return pl.pallas_call(
        _head_body,
        grid=(nk,),
        in_specs=[
            pl.BlockSpec((BATCH, NUM_SYMPTOMS), lambda j: (0, 0)),
            pl.BlockSpec((NUM_SYMPTOMS, HID), lambda j: (0, 0)),
            pl.BlockSpec((NUM_SYMPTOMS, HID), lambda j: (0, 0)),
            pl.BlockSpec((NUM_HERBS, HID), lambda j: (0, 0)),
            pl.BlockSpec((NUM_HERBS, HID), lambda j: (0, 0)),
            pl.BlockSpec((NUM_HERBS, HID), lambda j: (0, 0)),
            pl.BlockSpec((NUM_HERBS, BK), lambda j: (0, j)),
            pl.BlockSpec((1, BK), lambda j: (0, j)),
            pl.BlockSpec((BK, NUM_HERBS), lambda j: (j, 0)),
            pl.BlockSpec((1, NUM_HERBS), lambda j: (0, 0)),
        ],
        out_specs=pl.BlockSpec((BATCH, NUM_HERBS), lambda j: (0, 0)),
        out_shape=jax.ShapeDtypeStruct((BATCH, NUM_HERBS), jnp.float32),
        scratch_shapes=[pltpu.VMEM((BATCH, NUM_HERBS), jnp.float32)],
    )(x, sx, cxs, hx, cxh, hfh, w1, b1r, w2, b2r)


def _with_loops(ei, N):
    loops = jnp.arange(N, dtype=jnp.int32)
    return (jnp.concatenate([ei[0], loops]),
            jnp.concatenate([ei[1], loops]))


def _colsplit(a):
    half = a.shape[1] // 2
    return jnp.concatenate([a[:, :half], a[:, half:]], axis=0)


def kernel(x, herb_x, symptom_x, cross_x, hyper_x, params, herb_edge_index,
           symptom_edge_index, cross_edge_index, hyper_edge_index,
           hyper_edge_mapping):
    p = params

    def two_gats(xin, ei, p1, p2):
        N = xin.shape[0]
        src, dst = _with_loops(ei, N)
        h1 = _gat_layer(xin, src, dst, HEADS, p1)
        return _gat_layer(h1, src, dst, 1, p2)

    hx = two_gats(herb_x, herb_edge_index, p['herb_gat1'], p['herb_gat2'])
    sx = two_gats(symptom_x, symptom_edge_index, p['sym_gat1'], p['sym_gat2'])
    cx = two_gats(cross_x, cross_edge_index, p['cross_gat1'], p['cross_gat2'])

    node = hyper_edge_index[0]
    he = hyper_edge_index[1]
    zpad = jnp.zeros((_B,), jnp.int32)
    nodep = jnp.concatenate([node, zpad])
    hep = jnp.concatenate([he, zpad])
    nodeb = node.reshape(16, -1, _B)
    heb = he.reshape(16, -1, _B)

    xw1 = _mm(hyper_x, p['hyper1']['W'])
    hy1_2 = _hyper_layer(_colsplit(xw1), nodep, hep, nodeb, heb,
                         p['hyper1']['b'], True)
    hy1 = jnp.concatenate([hy1_2[:NUM_INGREDIENTS], hy1_2[NUM_INGREDIENTS:]],
                          axis=1)
    xw2 = _mm(hy1, p['hyper2']['W'])
    hy2_2 = _hyper_layer(_colsplit(xw2), nodep, hep, nodeb, heb,
                         p['hyper2']['b'], False)

    mapb = hyper_edge_mapping.reshape(16, -1, _B)
    hfh2 = _make_smean()(hy2_2, mapb)
    hfh = jnp.concatenate([hfh2[:NUM_HERBS], hfh2[NUM_HERBS:]], axis=1)

    return _head(x, sx, cx[:NUM_SYMPTOMS], hx, cx[NUM_SYMPTOMS:], hfh,
                 p['pred_W1'], p['pred_b1'], p['pred_W2'], p['pred_b2'])


# speedup vs baseline: 12.3506x; 1.0641x over previous
import functools

import jax
import jax.numpy as jnp
from jax import lax
from jax.experimental import pallas as pl
from jax.experimental.pallas import tpu as pltpu
from jax.experimental.pallas import tpu_sc as plsc

HID = 128
HEADS = 4
NUM_HERBS = 4096
NUM_SYMPTOMS = 2048
NUM_INGREDIENTS = 8192
NUM_HYPEREDGES = 2048
BATCH = 256

_B = 64

_SC_PARAMS = pltpu.CompilerParams(use_tc_tiling_on_sc=False,
                                  needs_layout_passes=False)


def _mesh():
    return plsc.VectorSubcoreMesh(core_axis_name="c", subcore_axis_name="s",
                                  num_cores=2, num_subcores=16)


def _mm_body(x_ref, w_ref, o_ref):
    o_ref[...] = jnp.dot(x_ref[...], w_ref[...],
                         preferred_element_type=jnp.float32)


def _mm(x, w):
    n, _ = x.shape
    m = w.shape[1]
    return pl.pallas_call(
        _mm_body,
        out_shape=jax.ShapeDtypeStruct((n, m), jnp.float32),
    )(x, w)


@functools.lru_cache(maxsize=None)
def _make_gat(N, Etot, NH, CW):
    B = _B
    E16 = Etot // 16
    NB = E16 // B
    NR = N // 16
    NRB = NR // B
    NP = max(CW // 64, 1)
    PW = CW // NP
    NCB = PW // 16
    HL16 = (CW // NH) // 16

    @functools.partial(
        pl.kernel,
        out_type=jax.ShapeDtypeStruct((2 * NP * N, PW), jnp.float32),
        mesh=_mesh(),
        compiler_params=_SC_PARAMS,
        scratch_types=[
            pltpu.VMEM((N * NH,), jnp.float32),
            pltpu.VMEM((N * NH,), jnp.float32),
            pltpu.VMEM((NB, B), jnp.int32),
            pltpu.VMEM((NH * E16,), jnp.float32),
            pltpu.VMEM((B, 16), jnp.float32),
            pltpu.VMEM((N * NH,), jnp.float32),
            pltpu.VMEM((NR, 16), jnp.float32),
            [pltpu.VMEM((B, PW), jnp.float32)] * 3,
            [pltpu.VMEM((B,), jnp.int32)] * 3,
            pltpu.VMEM((PW,), jnp.float32),
            pltpu.VMEM((NH * B,), jnp.float32),
            pltpu.VMEM_SHARED((N, 16), jnp.float32),
            pltpu.VMEM_SHARED((N * NH,), jnp.float32),
            pltpu.VMEM_SHARED((N, PW), jnp.float32),
            [pltpu.SemaphoreType.DMA] * 3,
        ],
    )
    def gat(h_hbm, asc_hbm, adc_hbm, srcp_hbm, dstb_hbm, b_hbm, out_hbm,
            asrc_t, adst_t, dstb, exl, exrow, rden_t, denrow, rowbufs,
            srcbufs, bias_t, alpha_t, den_sh, rden_sh, acc_sh, sems):
        rows_a = rowbufs[0]
        c = lax.axis_index("c")
        s = lax.axis_index("s")
        iota = lax.iota(jnp.int32, 16)
        zero16 = jnp.zeros((16,), jnp.float32)
        toff = s * E16
        rbase = s * NR

        pltpu.sync_copy(asc_hbm.at[c], asrc_t)
        pltpu.sync_copy(adc_hbm.at[c], adst_t)
        pltpu.sync_copy(dstb_hbm.at[s], dstb)

        for i in range(B):
            exrow[i, :] = zero16

        def _zrow(i, carry):
            ii = jnp.full((16,), i, jnp.int32)
            for j in range(NCB):
                plsc.store_scatter(rows_a, [ii, j * 16 + iota], zero16)
            return carry
        lax.fori_loop(0, B, _zrow, 0)

        for q in range(NRB):
            pltpu.sync_copy(exrow, den_sh.at[pl.ds(rbase + q * B, B)])
        plsc.subcore_barrier()

        def _p1(blk, carry):
            pltpu.sync_copy(srcp_hbm.at[pl.ds(toff + blk * B, B)], srcbufs[0])
            ebase = blk * B
            for sub in range(4):
                off = sub * 16 + iota
                s16 = plsc.load_gather(srcbufs[0], [off])
                d16 = plsc.load_gather(
                    dstb, [jnp.full((16,), 0, jnp.int32) + blk, off])
                for hh in range(NH):
                    av = plsc.load_gather(asrc_t, [s16 * NH + hh])
                    bv = plsc.load_gather(adst_t, [d16 * NH + hh])
                    e = av + bv
                    e = jnp.where(e > 0, e, e * jnp.float32(0.2))
                    ex = jnp.exp(e)
                    plsc.store_scatter(
                        exl, [ebase + (hh * E16 + sub * 16) + iota], ex)
                    plsc.store_scatter(
                        exrow, [off, jnp.full((16,), hh, jnp.int32)], ex)
            pltpu.sync_copy(exrow, den_sh.at[dstb.at[blk]], add=True)
            return carry
        lax.fori_loop(0, NB, _p1, 0)
        plsc.subcore_barrier()

        pltpu.sync_copy(den_sh.at[pl.ds(rbase, NR)], denrow)
        for i in range(NR // 16):
            for hh in range(NH):
                d = plsc.load_gather(
                    denrow, [i * 16 + iota, jnp.full((16,), hh, jnp.int32)])
                r = 1.0 / (d + jnp.float32(1e-16))
                plsc.store_scatter(rden_t, [(i * 16 + iota) * NH + hh], r)
        pltpu.sync_copy(rden_t.at[pl.ds(0, NR * NH)],
                        rden_sh.at[pl.ds(rbase * NH, NR * NH)])
        plsc.subcore_barrier()
        pltpu.sync_copy(rden_sh, rden_t)

        for ps in range(NP):
            base = c * NP + ps
            gN = base * N

            def _azrow(i, carry):
                ii = jnp.full((16,), i, jnp.int32)
                for j in range(NCB):
                    plsc.store_scatter(rows_a, [ii, j * 16 + iota], zero16)
                return carry
            lax.fori_loop(0, B, _azrow, 0)
            for q in range(NRB):
                pltpu.sync_copy(rows_a, acc_sh.at[pl.ds(rbase + q * B, B)])
            plsc.subcore_barrier()

            def _loadsrc(blk, sref):
                pltpu.sync_copy(srcp_hbm.at[pl.ds(toff + blk * B, B)], sref)
                for q in range(B // 16):
                    v = plsc.load_gather(sref, [q * 16 + iota])
                    plsc.store_scatter(sref, [q * 16 + iota], v + gN)

            hh_pass = (ps * NCB) // HL16

            def _scale_scatter(blk, rows):
                hh = hh_pass
                for sub in range(4):
                    off = sub * 16 + iota
                    d16 = plsc.load_gather(
                        dstb, [jnp.full((16,), 0, jnp.int32) + blk, off])
                    ex16 = plsc.load_gather(
                        exl, [blk * B + (hh * E16 + sub * 16) + iota])
                    rd16 = plsc.load_gather(rden_t, [d16 * NH + hh])
                    plsc.store_scatter(
                        alpha_t, [sub * 16 + iota], ex16 * rd16)

                def _srow(i, carry):
                    ii = jnp.full((16,), i, jnp.int32)
                    bc = plsc.load_gather(
                        alpha_t, [jnp.full((16,), 0, jnp.int32) + i])
                    for j in range(NCB):
                        v = plsc.load_gather(rows, [ii, j * 16 + iota])
                        plsc.store_scatter(rows, [ii, j * 16 + iota], v * bc)
                    return carry
                lax.fori_loop(0, B, _srow, 0)
                pltpu.sync_copy(rows, acc_sh.at[dstb.at[blk]], add=True)

            NPIPE = 3
            for b in range(NPIPE):
                _loadsrc(b, srcbufs[b])
                pltpu.async_copy(h_hbm.at[srcbufs[b]], rowbufs[b], sems[b])

            def _p2(q, carry):
                for b in range(NPIPE):
                    blk = q * NPIPE + b
                    pltpu.make_async_copy(
                        h_hbm.at[pl.ds(0, B)], rowbufs[b], sems[b]).wait()
                    _scale_scatter(blk, rowbufs[b])
                    _loadsrc(blk + NPIPE, srcbufs[b])
                    pltpu.async_copy(
                        h_hbm.at[srcbufs[b]], rowbufs[b], sems[b])
                return carry
            lax.fori_loop(0, NB // NPIPE, _p2, 0)
            for b in range(NPIPE):
                pltpu.make_async_copy(
                    h_hbm.at[pl.ds(0, B)], rowbufs[b], sems[b]).wait()
            plsc.subcore_barrier()

            pltpu.sync_copy(b_hbm.at[base], bias_t)
            for q in range(NRB):
                row0 = rbase + q * B
                pltpu.sync_copy(acc_sh.at[pl.ds(row0, B)], rows_a)

                def _orow(i, carry):
                    ii = jnp.full((16,), i, jnp.int32)
                    for j in range(NCB):
                        v = plsc.load_gather(rows_a, [ii, j * 16 + iota])
                        v = v + bias_t[pl.ds(j * 16, 16)]
                        v = jnp.where(v > 0, v, jnp.exp(v) - 1.0)
                        plsc.store_scatter(rows_a, [ii, j * 16 + iota], v)
                    return carry
                lax.fori_loop(0, B, _orow, 0)
                pltpu.sync_copy(rows_a, out_hbm.at[pl.ds(gN + row0, B)])
            plsc.subcore_barrier()

    return gat


def _gat_layer(x, src, dst, H, p):
    N = x.shape[0]
    Etot = src.shape[0]
    W, a_src, a_dst, b = p['W'], p['a_src'], p['a_dst'], p['b']
    din = W.shape[0]
    Asrc = jnp.einsum('dhk,hk->dh', W.reshape(din, H, HID), a_src[0])
    Adst = jnp.einsum('dhk,hk->dh', W.reshape(din, H, HID), a_dst[0])
    pad = jnp.zeros((din, 128 - 2 * H), jnp.float32)
    Wext = jnp.concatenate([W, Asrc, Adst, pad], axis=1)
    ho = _mm(x, Wext)
    h = ho[:, :H * HID]
    asrc = ho[:, H * HID:H * HID + H]
    adst = ho[:, H * HID + H:H * HID + 2 * H]

    CW = (H * HID) // 2
    NH = max(H // 2, 1)
    NP = max(CW // 64, 1)
    PW = CW // NP
    nparts = 2 * NP
    h_sc = jnp.concatenate(
        [h[:, i * PW:(i + 1) * PW] for i in range(nparts)], axis=0)
    if H > 1:
        asc = jnp.stack([asrc[:, :NH].reshape(-1), asrc[:, NH:].reshape(-1)])
        adc = jnp.stack([adst[:, :NH].reshape(-1), adst[:, NH:].reshape(-1)])
    else:
        asc = jnp.stack([asrc[:, 0], asrc[:, 0]])
        adc = jnp.stack([adst[:, 0], adst[:, 0]])
    b_sc = jnp.stack([b[i * PW:(i + 1) * PW] for i in range(nparts)])
    srcp = jnp.concatenate([src, jnp.zeros((8 * _B,), jnp.int32)])
    dstb = dst.reshape(16, -1, _B)

    out2 = _make_gat(N, Etot, NH, CW)(h_sc, asc, adc, srcp, dstb, b_sc)
    return jnp.concatenate(
        [out2[i * N:(i + 1) * N] for i in range(nparts)], axis=1)


@functools.lru_cache(maxsize=None)
def _make_hyper1():
    B = _B
    NI = NUM_INGREDIENTS
    NE = NUM_HYPEREDGES
    E = 65536
    E16 = E // 16
    NB = E16 // B
    NRI = NI // 16
    NRE = NE // 16
    CW = 64
    NCB = CW // 16

    @functools.partial(
        pl.kernel,
        out_type=[jax.ShapeDtypeStruct((2 * NE, CW), jnp.float32),
                  jax.ShapeDtypeStruct((2 * NI,), jnp.float32)],
        mesh=_mesh(),
        compiler_params=_SC_PARAMS,
        scratch_types=[
            pltpu.VMEM((NB, B), jnp.int32),
            pltpu.VMEM((NB, B), jnp.int32),
            [pltpu.VMEM((B, CW), jnp.float32)] * 4,
            [pltpu.VMEM((B,), jnp.int32)] * 4,
            pltpu.VMEM((B, 16), jnp.float32),
            pltpu.VMEM((NRE, 16), jnp.float32),
            pltpu.VMEM((NRI, 16), jnp.float32),
            pltpu.VMEM((NRI,), jnp.float32),
            pltpu.VMEM_SHARED((NE, CW), jnp.float32),
            pltpu.VMEM_SHARED((NI, 16), jnp.float32),
            pltpu.VMEM_SHARED((NE, 16), jnp.float32),
            [pltpu.SemaphoreType.DMA] * 4,
        ],
    )
    def hyper1(xw_hbm, nodep_hbm, nodeb_hbm, heb_hbm, msg_hbm, dinv_hbm,
               nodeb, heb, rowbufs, idxbufs, onesrow, degErow,
               degNrow, dinv_v, msg_sh, degN_sh, degE_sh, sems):
        rows_a = rowbufs[0]
        c = lax.axis_index("c")
        s = lax.axis_index("s")
        iota = lax.iota(jnp.int32, 16)
        zero16 = jnp.zeros((16,), jnp.float32)
        one0 = jnp.where(iota == 0, jnp.float32(1.0), jnp.float32(0.0))
        toff = s * E16

        pltpu.sync_copy(nodeb_hbm.at[s], nodeb)
        pltpu.sync_copy(heb_hbm.at[s], heb)

        for i in range(B):
            onesrow[i, :] = zero16

        def _zrow(i, carry):
            ii = jnp.full((16,), i, jnp.int32)
            for j in range(NCB):
                plsc.store_scatter(rows_a, [ii, j * 16 + iota], zero16)
            return carry
        lax.fori_loop(0, B, _zrow, 0)

        for q in range(NRI // B):
            pltpu.sync_copy(onesrow, degN_sh.at[pl.ds(s * NRI + q * B, B)])
        for q in range(NRE // B):
            pltpu.sync_copy(onesrow, degE_sh.at[pl.ds(s * NRE + q * B, B)])
            pltpu.sync_copy(rows_a, msg_sh.at[pl.ds(s * NRE + q * B, B)])
        for i in range(B):
            onesrow[i, :] = one0
        plsc.subcore_barrier()

        def _loadidx(hbm, blk, sref, shift):
            pltpu.sync_copy(hbm.at[pl.ds(toff + blk * B, B)], sref)
            for q in range(B // 16):
                v = plsc.load_gather(sref, [q * 16 + iota])
                plsc.store_scatter(sref, [q * 16 + iota], v + shift)

        NPIPE = 4
        for b in range(NPIPE):
            _loadidx(nodep_hbm, b, idxbufs[b], c * NI)
            pltpu.async_copy(xw_hbm.at[idxbufs[b]], rowbufs[b], sems[b])

        def _pa(q, carry):
            for b in range(NPIPE):
                blk = q * NPIPE + b
                pltpu.make_async_copy(
                    xw_hbm.at[pl.ds(0, B)], rowbufs[b], sems[b]).wait()
                pltpu.sync_copy(rowbufs[b], msg_sh.at[heb.at[blk]], add=True)
                pltpu.sync_copy(onesrow, degN_sh.at[nodeb.at[blk]], add=True)
                pltpu.sync_copy(onesrow, degE_sh.at[heb.at[blk]], add=True)
                _loadidx(nodep_hbm, blk + NPIPE, idxbufs[b], c * NI)
                pltpu.async_copy(xw_hbm.at[idxbufs[b]], rowbufs[b], sems[b])
            return carry
        lax.fori_loop(0, NB // NPIPE, _pa, 0)
        for b in range(NPIPE):
            pltpu.make_async_copy(
                xw_hbm.at[pl.ds(0, B)], rowbufs[b], sems[b]).wait()
        plsc.subcore_barrier()

        pltpu.sync_copy(degE_sh.at[pl.ds(s * NRE, NRE)], degErow)
        for q in range(NRE // B):
            pltpu.sync_copy(msg_sh.at[pl.ds(s * NRE + q * B, B)], rows_a)

            def _mrow(i, carry):
                ii = jnp.full((16,), i, jnp.int32)
                d = plsc.load_gather(
                    degErow,
                    [jnp.full((16,), q * B, jnp.int32) + i,
                     jnp.zeros((16,), jnp.int32)])
                binv = jnp.where(d > 0, 1.0 / jnp.maximum(d, 1.0),
                                 jnp.float32(0.0))
                for j in range(NCB):
                    v = plsc.load_gather(rows_a, [ii, j * 16 + iota])
                    plsc.store_scatter(rows_a, [ii, j * 16 + iota], v * binv)
                return carry
            lax.fori_loop(0, B, _mrow, 0)
            pltpu.sync_copy(
                rows_a, msg_hbm.at[pl.ds(c * NE + s * NRE + q * B, B)])

        pltpu.sync_copy(degN_sh.at[pl.ds(s * NRI, NRI)], degNrow)
        for i in range(NRI // 16):
            d = plsc.load_gather(degNrow,
                                 [i * 16 + iota, jnp.zeros((16,), jnp.int32)])
            dinv = jnp.where(d > 0, 1.0 / jnp.maximum(d, 1.0),
                             jnp.float32(0.0))
            plsc.store_scatter(dinv_v, [i * 16 + iota], dinv)
        pltpu.sync_copy(dinv_v, dinv_hbm.at[pl.ds(c * NI + s * NRI, NRI)])

    return hyper1


@functools.lru_cache(maxsize=None)
def _make_hyper2(relu):
    B = _B
    NI = NUM_INGREDIENTS
    NE = NUM_HYPEREDGES
    E = 65536
    E16 = E // 16
    NB = E16 // B
    NRI = NI // 16
    CW = 64
    NCB = CW // 16

    @functools.partial(
        pl.kernel,
        out_type=jax.ShapeDtypeStruct((2 * NI, CW), jnp.float32),
        mesh=_mesh(),
        compiler_params=_SC_PARAMS,
        scratch_types=[
            pltpu.VMEM((NB, B), jnp.int32),
            [pltpu.VMEM((B, CW), jnp.float32)] * 4,
            [pltpu.VMEM((B,), jnp.int32)] * 4,
            pltpu.VMEM((NRI,), jnp.float32),
            pltpu.VMEM((CW,), jnp.float32),
            pltpu.VMEM_SHARED((NI, CW), jnp.float32),
            [pltpu.SemaphoreType.DMA] * 4,
        ],
    )
    def hyper2(msg_hbm, hep_hbm, nodeb_hbm, dinv_hbm, b_hbm, out_hbm,
               nodeb, rowbufs, idxbufs, dinv_v, bias_t, acc_sh, sems):
        rows_a = rowbufs[0]
        c = lax.axis_index("c")
        s = lax.axis_index("s")
        iota = lax.iota(jnp.int32, 16)
        zero16 = jnp.zeros((16,), jnp.float32)
        toff = s * E16

        pltpu.sync_copy(nodeb_hbm.at[s], nodeb)
        pltpu.sync_copy(b_hbm.at[c], bias_t)
        pltpu.sync_copy(dinv_hbm.at[pl.ds(c * NI + s * NRI, NRI)], dinv_v)

        def _zrow(i, carry):
            ii = jnp.full((16,), i, jnp.int32)
            for j in range(NCB):
                plsc.store_scatter(rows_a, [ii, j * 16 + iota], zero16)
            return carry
        lax.fori_loop(0, B, _zrow, 0)
        for q in range(NRI // B):
            pltpu.sync_copy(rows_a, acc_sh.at[pl.ds(s * NRI + q * B, B)])
        plsc.subcore_barrier()

        def _loadidx(hbm, blk, sref, shift):
            pltpu.sync_copy(hbm.at[pl.ds(toff + blk * B, B)], sref)
            for q in range(B // 16):
                v = plsc.load_gather(sref, [q * 16 + iota])
                plsc.store_scatter(sref, [q * 16 + iota], v + shift)

        NPIPE = 4
        for b in range(NPIPE):
            _loadidx(hep_hbm, b, idxbufs[b], c * NE)
            pltpu.async_copy(msg_hbm.at[idxbufs[b]], rowbufs[b], sems[b])

        def _pb(q, carry):
            for b in range(NPIPE):
                blk = q * NPIPE + b
                pltpu.make_async_copy(
                    msg_hbm.at[pl.ds(0, B)], rowbufs[b], sems[b]).wait()
                pltpu.sync_copy(rowbufs[b], acc_sh.at[nodeb.at[blk]],
                                add=True)
                _loadidx(hep_hbm, blk + NPIPE, idxbufs[b], c * NE)
                pltpu.async_copy(msg_hbm.at[idxbufs[b]], rowbufs[b], sems[b])
            return carry
        lax.fori_loop(0, NB // NPIPE, _pb, 0)
        for b in range(NPIPE):
            pltpu.make_async_copy(
                msg_hbm.at[pl.ds(0, B)], rowbufs[b], sems[b]).wait()
        plsc.subcore_barrier()

        for q in range(NRI // B):
            row0 = s * NRI + q * B
            pltpu.sync_copy(acc_sh.at[pl.ds(row0, B)], rows_a)

            def _orow(i, carry):
                ii = jnp.full((16,), i, jnp.int32)
                dinv = plsc.load_gather(
                    dinv_v, [jnp.full((16,), q * B, jnp.int32) + i])
                for j in range(NCB):
                    v = plsc.load_gather(rows_a, [ii, j * 16 + iota])
                    v = v * dinv + bias_t[pl.ds(j * 16, 16)]
                    if relu:
                        v = jnp.maximum(v, jnp.float32(0.0))
                    plsc.store_scatter(rows_a, [ii, j * 16 + iota], v)
                return carry
            lax.fori_loop(0, B, _orow, 0)
            pltpu.sync_copy(rows_a, out_hbm.at[pl.ds(c * NI + row0, B)])

    return hyper2


def _hyper_layer(xw_split, nodep, hep, nodeb, heb, b, relu):
    b_sc = jnp.stack([b[:64], b[64:]])
    msg, dinv = _make_hyper1()(xw_split, nodep, nodeb, heb)
    return _make_hyper2(relu)(msg, hep, nodeb, dinv, b_sc)


@functools.lru_cache(maxsize=None)
def _make_smean():
    B = _B
    NS = NUM_INGREDIENTS
    ND = NUM_HERBS
    S16 = NS // 16
    NB = S16 // B
    NRD = ND // 16
    CW = 64
    NCB = CW // 16

    @functools.partial(
        pl.kernel,
        out_type=jax.ShapeDtypeStruct((2 * ND, CW), jnp.float32),
        mesh=_mesh(),
        compiler_params=_SC_PARAMS,
        scratch_types=[
            pltpu.VMEM((NB, B), jnp.int32),
            pltpu.VMEM((B, CW), jnp.float32),
            pltpu.VMEM((B, 16), jnp.float32),
            pltpu.VMEM((NRD, 16), jnp.float32),
            pltpu.VMEM_SHARED((ND, 16), jnp.float32),
            pltpu.VMEM_SHARED((ND, CW), jnp.float32),
        ],
    )
    def smean(src_hbm, mapb_hbm, out_hbm, mapb, rows_a, onesrow, degrow,
              deg_sh, acc_sh):
        c = lax.axis_index("c")
        s = lax.axis_index("s")
        iota = lax.iota(jnp.int32, 16)
        zero16 = jnp.zeros((16,), jnp.float32)
        one0 = jnp.where(iota == 0, jnp.float32(1.0), jnp.float32(0.0))

        pltpu.sync_copy(mapb_hbm.at[s], mapb)
        for i in range(B):
            onesrow[i, :] = zero16

        def _zrow(i, carry):
            ii = jnp.full((16,), i, jnp.int32)
            for j in range(NCB):
                plsc.store_scatter(rows_a, [ii, j * 16 + iota], zero16)
            return carry
        lax.fori_loop(0, B, _zrow, 0)

        for q in range(NRD // B):
            pltpu.sync_copy(onesrow, deg_sh.at[pl.ds(s * NRD + q * B, B)])
            pltpu.sync_copy(rows_a, acc_sh.at[pl.ds(s * NRD + q * B, B)])
        for i in range(B):
            onesrow[i, :] = one0
        plsc.subcore_barrier()

        def _pa(blk, carry):
            pltpu.sync_copy(
                src_hbm.at[pl.ds(c * NS + s * S16 + blk * B, B)], rows_a)
            pltpu.sync_copy(rows_a, acc_sh.at[mapb.at[blk]], add=True)
            pltpu.sync_copy(onesrow, deg_sh.at[mapb.at[blk]], add=True)
            return carry
        lax.fori_loop(0, NB, _pa, 0)
        plsc.subcore_barrier()

        pltpu.sync_copy(deg_sh.at[pl.ds(s * NRD, NRD)], degrow)
        for q in range(NRD // B):
            row0 = s * NRD + q * B
            pltpu.sync_copy(acc_sh.at[pl.ds(row0, B)], rows_a)

            def _orow(i, carry):
                ii = jnp.full((16,), i, jnp.int32)
                d = plsc.load_gather(
                    degrow,
                    [jnp.full((16,), q * B, jnp.int32) + i,
                     jnp.zeros((16,), jnp.int32)])
                rinv = 1.0 / jnp.maximum(d, jnp.float32(1.0))
                for j in range(NCB):
                    v = plsc.load_gather(rows_a, [ii, j * 16 + iota])
                    plsc.store_scatter(rows_a, [ii, j * 16 + iota], v * rinv)
                return carry
            lax.fori_loop(0, B, _orow, 0)
            pltpu.sync_copy(rows_a, out_hbm.at[pl.ds(c * ND + row0, B)])

    return smean


def _head_body(x_ref, sx_ref, cxs_ref, hx_ref, cxh_ref, hfh_ref, w1_ref,
               b1_ref, w2_ref, b2_ref, out_ref, logits_ref):
    j = pl.program_id(0)

    @pl.when(j == 0)
    def _():
        fs = sx_ref[...] + cxs_ref[...]
        fh = hx_ref[...] + cxh_ref[...] + hfh_ref[...]
        semb = jnp.dot(x_ref[...], fs, preferred_element_type=jnp.float32)
        logits_ref[...] = lax.dot_general(
            semb, fh, (((1,), (1,)), ((), ())),
            preferred_element_type=jnp.float32)
        out_ref[...] = jnp.broadcast_to(b2_ref[...], out_ref.shape)

    hid = jnp.dot(logits_ref[...], w1_ref[...],
                  preferred_element_type=jnp.float32) + b1_ref[...]
    hid = jnp.maximum(hid, 0.0)
    out_ref[...] += jnp.dot(hid, w2_ref[...],
                            preferred_element_type=jnp.float32)


def _head(x, sx, cxs, hx, cxh, hfh, w1, b1, w2, b2):
    BK = 512
    nk = w1.shape[1] // BK
    b1r = b1.reshape(1, -1)
    b2r = b2.reshape(1, -1)
    return pl.pallas_call(
        _head_body,
        grid=(nk,),
        in_specs=[
            pl.BlockSpec((BATCH, NUM_SYMPTOMS), lambda j: (0, 0)),
            pl.BlockSpec((NUM_SYMPTOMS, HID), lambda j: (0, 0)),
            pl.BlockSpec((NUM_SYMPTOMS, HID), lambda j: (0, 0)),
            pl.BlockSpec((NUM_HERBS, HID), lambda j: (0, 0)),
            pl.BlockSpec((NUM_HERBS, HID), lambda j: (0, 0)),
            pl.BlockSpec((NUM_HERBS, HID), lambda j: (0, 0)),
            pl.BlockSpec((NUM_HERBS, BK), lambda j: (0, j)),
            pl.BlockSpec((1, BK), lambda j: (0, j)),
            pl.BlockSpec((BK, NUM_HERBS), lambda j: (j, 0)),
            pl.BlockSpec((1, NUM_HERBS), lambda j: (0, 0)),
        ],
        out_specs=pl.BlockSpec((BATCH, NUM_HERBS), lambda j: (0, 0)),
        out_shape=jax.ShapeDtypeStruct((BATCH, NUM_HERBS), jnp.float32),
        scratch_shapes=[pltpu.VMEM((BATCH, NUM_HERBS), jnp.float32)],
    )(x, sx, cxs, hx, cxh, hfh, w1, b1r, w2, b2r)


def _with_loops(ei, N):
    loops = jnp.arange(N, dtype=jnp.int32)
    return (jnp.concatenate([ei[0], loops]),
            jnp.concatenate([ei[1], loops]))


def _colsplit(a):
    half = a.shape[1] // 2
    return jnp.concatenate([a[:, :half], a[:, half:]], axis=0)


def kernel(x, herb_x, symptom_x, cross_x, hyper_x, params, herb_edge_index,
           symptom_edge_index, cross_edge_index, hyper_edge_index,
           hyper_edge_mapping):
    p = params

    def two_gats(xin, ei, p1, p2):
        N = xin.shape[0]
        src, dst = _with_loops(ei, N)
        h1 = _gat_layer(xin, src, dst, HEADS, p1)
        return _gat_layer(h1, src, dst, 1, p2)

    hx = two_gats(herb_x, herb_edge_index, p['herb_gat1'], p['herb_gat2'])
    sx = two_gats(symptom_x, symptom_edge_index, p['sym_gat1'], p['sym_gat2'])
    cx = two_gats(cross_x, cross_edge_index, p['cross_gat1'], p['cross_gat2'])

    node = hyper_edge_index[0]
    he = hyper_edge_index[1]
    zpad = jnp.zeros((8 * _B,), jnp.int32)
    nodep = jnp.concatenate([node, zpad])
    hep = jnp.concatenate([he, zpad])
    nodeb = node.reshape(16, -1, _B)
    heb = he.reshape(16, -1, _B)

    xw1 = _mm(hyper_x, p['hyper1']['W'])
    hy1_2 = _hyper_layer(_colsplit(xw1), nodep, hep, nodeb, heb,
                         p['hyper1']['b'], True)
    hy1 = jnp.concatenate([hy1_2[:NUM_INGREDIENTS], hy1_2[NUM_INGREDIENTS:]],
                          axis=1)
    xw2 = _mm(hy1, p['hyper2']['W'])
    hy2_2 = _hyper_layer(_colsplit(xw2), nodep, hep, nodeb, heb,
                         p['hyper2']['b'], False)

    mapb = hyper_edge_mapping.reshape(16, -1, _B)
    hfh2 = _make_smean()(hy2_2, mapb)
    hfh = jnp.concatenate([hfh2[:NUM_HERBS], hfh2[NUM_HERBS:]], axis=1)

    return _head(x, sx, cx[:NUM_SYMPTOMS], hx, cx[NUM_SYMPTOMS:], hfh,
                 p['pred_W1'], p['pred_b1'], p['pred_W2'], p['pred_b2'])
